# baseline (speedup 1.0000x reference)
import functools

import jax
import jax.numpy as jnp
from jax import lax
from jax.experimental import pallas as pl
from jax.experimental.pallas import tpu as pltpu
from jax.experimental.pallas import tpu_sc as plsc

_NC, _NS = 2, 16
_NW = _NC * _NS
_CHUNK = 200
_NBUF = 4
_REP = 4
_LANES = 16


def kernel(ids, table):
    B, T = ids.shape
    V, D = table.shape
    N = B * T
    assert N % (_NW * _NBUF * _CHUNK) == 0
    b_per_w = N // _NW
    n_chunks = b_per_w // _CHUNK
    n4 = n_chunks // _NBUF
    flat_ids = ids.reshape(N)
    table128 = jnp.tile(jnp.pad(table, ((0, 0), (0, 128 - D))), (_REP, 1))

    mesh = plsc.VectorSubcoreMesh(core_axis_name="c", subcore_axis_name="s")

    @functools.partial(
        pl.kernel,
        mesh=mesh,
        out_type=jax.ShapeDtypeStruct((N, 128), table.dtype),
        scratch_types=[
            pltpu.VMEM((b_per_w,), jnp.int32),
        ] + [pltpu.VMEM((_CHUNK, 128), jnp.float32)] * _NBUF
          + [pltpu.SemaphoreType.DMA] * (2 * _NBUF),
    )
    def k(table_hbm, idx_hbm, out_hbm, idx_all, *bufs_and_sems):
        rows = bufs_and_sems[:_NBUF]
        gsem = bufs_and_sems[_NBUF:2 * _NBUF]
        ssem = bufs_and_sems[2 * _NBUF:]

        wid = lax.axis_index("s") * _NC + lax.axis_index("c")
        base = wid * b_per_w
        pltpu.sync_copy(idx_hbm.at[pl.ds(base, b_per_w)], idx_all)

        rep_off = (wid % _REP) * V

        @pl.loop(0, b_per_w, step=_LANES)
        def _(r):
            sl = pl.ds(r, _LANES)
            idx_all.at[sl][...] = idx_all.at[sl][...] + rep_off

        def gather_desc(i, b):
            return pltpu.make_async_copy(
                table_hbm.at[idx_all.at[pl.ds(i * _CHUNK, _CHUNK)]],
                rows[b], gsem[b])

        def store_desc(i, b):
            return pltpu.make_async_copy(
                rows[b], out_hbm.at[pl.ds(base + i * _CHUNK, _CHUNK)], ssem[b])

        gather_desc(0, 0).start()
        gather_desc(1, 1).start()

        @pl.loop(0, n4)
        def _(j):
            for b in range(_NBUF):
                i = _NBUF * j + b
                b2 = (b + 2) % _NBUF

                if b < 2:
                    @pl.when(j > 0)
                    def _():
                        store_desc(i - 2, b2).wait()

                    gather_desc(i + 2, b2).start()
                else:
                    store_desc(i - 2, b2).wait()

                    @pl.when(j < n4 - 1)
                    def _():
                        gather_desc(i + 2, b2).start()

                gather_desc(i, b).wait()
                store_desc(i, b).start()

        store_desc(n_chunks - 2, (n_chunks - 2) % _NBUF).wait()
        store_desc(n_chunks - 1, (n_chunks - 1) % _NBUF).wait()

    out = k(table128, flat_ids)
    return out[:, :D].reshape(B, T, D)


# speedup vs baseline: 1.0021x; 1.0021x over previous
import functools

import jax
import jax.numpy as jnp
from jax import lax
from jax.experimental import pallas as pl
from jax.experimental.pallas import tpu as pltpu
from jax.experimental.pallas import tpu_sc as plsc

_NC, _NS = 2, 16
_NW = _NC * _NS
_CHUNK = 160
_NBUF = 4
_REP = 4
_LANES = 16


def kernel(ids, table):
    B, T = ids.shape
    V, D = table.shape
    N = B * T
    assert N % (_NW * _NBUF * _CHUNK) == 0
    b_per_w = N // _NW
    n_chunks = b_per_w // _CHUNK
    n4 = n_chunks // _NBUF
    flat_ids = ids.reshape(N)
    table128 = jnp.tile(jnp.pad(table, ((0, 0), (0, 128 - D))), (_REP, 1))

    mesh = plsc.VectorSubcoreMesh(core_axis_name="c", subcore_axis_name="s")

    @functools.partial(
        pl.kernel,
        mesh=mesh,
        out_type=jax.ShapeDtypeStruct((N, 128), table.dtype),
        scratch_types=[
            pltpu.VMEM((b_per_w,), jnp.int32),
        ] + [pltpu.VMEM((_CHUNK, 128), jnp.float32)] * _NBUF
          + [pltpu.SemaphoreType.DMA] * (2 * _NBUF),
    )
    def k(table_hbm, idx_hbm, out_hbm, idx_all, *bufs_and_sems):
        rows = bufs_and_sems[:_NBUF]
        gsem = bufs_and_sems[_NBUF:2 * _NBUF]
        ssem = bufs_and_sems[2 * _NBUF:]

        wid = lax.axis_index("s") * _NC + lax.axis_index("c")
        base = wid * b_per_w
        pltpu.sync_copy(idx_hbm.at[pl.ds(base, b_per_w)], idx_all)

        rep_off = (wid % _REP) * V

        @pl.loop(0, b_per_w, step=_LANES)
        def _(r):
            sl = pl.ds(r, _LANES)
            idx_all.at[sl][...] = idx_all.at[sl][...] + rep_off

        def gather_desc(i, b):
            return pltpu.make_async_copy(
                table_hbm.at[idx_all.at[pl.ds(i * _CHUNK, _CHUNK)]],
                rows[b], gsem[b])

        def store_desc(i, b):
            return pltpu.make_async_copy(
                rows[b], out_hbm.at[pl.ds(base + i * _CHUNK, _CHUNK)], ssem[b])

        gather_desc(0, 0).start()
        gather_desc(1, 1).start()

        @pl.loop(0, n4)
        def _(j):
            for b in range(_NBUF):
                i = _NBUF * j + b
                b2 = (b + 2) % _NBUF

                if b < 2:
                    @pl.when(j > 0)
                    def _():
                        store_desc(i - 2, b2).wait()

                    gather_desc(i + 2, b2).start()
                else:
                    store_desc(i - 2, b2).wait()

                    @pl.when(j < n4 - 1)
                    def _():
                        gather_desc(i + 2, b2).start()

                gather_desc(i, b).wait()
                store_desc(i, b).start()

        store_desc(n_chunks - 2, (n_chunks - 2) % _NBUF).wait()
        store_desc(n_chunks - 1, (n_chunks - 1) % _NBUF).wait()

    out = k(table128, flat_ids)
    return out[:, :D].reshape(B, T, D)
